# Initial kernel scaffold; baseline (speedup 1.0000x reference)
import jax
import jax.numpy as jnp
from jax.experimental import pallas as pl


def kernel(inp, indices, values, bias):
    raise NotImplementedError("write your pallas kernel here")


# speedup vs baseline: 2.8527x; 2.8527x over previous
import functools

import jax
import jax.numpy as jnp
from jax import lax
from jax.experimental import pallas as pl
from jax.experimental.pallas import tpu as pltpu
from jax.experimental.pallas import tpu_sc as plsc

OUT_F = 16384
IN_F = 16384
BATCH = 256

NC = 2
NS = 16
LANES = 16
CHUNK = 128
BCHUNK = 64
ROWS_PER_TILE = OUT_F // NS


def _splat(v16, l):
    idx = jnp.full((LANES, 1), l, jnp.int32)
    return lax.gather(
        v16, idx,
        dimension_numbers=lax.GatherDimensionNumbers(
            offset_dims=(), collapsed_slice_dims=(0,), start_index_map=(0,)),
        slice_sizes=(1,),
        mode=lax.GatherScatterMode.PROMISE_IN_BOUNDS)


def _make_sc_call(n_iters: int):
    mesh = plsc.VectorSubcoreMesh(core_axis_name="c", subcore_axis_name="s")

    @functools.partial(
        pl.kernel,
        out_type=jax.ShapeDtypeStruct((4, OUT_F, BCHUNK), jnp.float32),
        mesh=mesh,
        compiler_params=pltpu.CompilerParams(use_tc_tiling_on_sc=False),
        scratch_types=[
            pltpu.VMEM((n_iters, CHUNK), jnp.int32),
            pltpu.VMEM((n_iters, CHUNK), jnp.int32),
            pltpu.VMEM((n_iters, CHUNK), jnp.float32),
            pltpu.VMEM((CHUNK, BCHUNK), jnp.float32),
            pltpu.VMEM_SHARED((OUT_F, BCHUNK), jnp.float32),
        ],
    )
    def sc_call(x_h, cols_h, rows_h, vals_h, bias_h, out_h,
                cols_vm, rows_vm, vals_vm, g_vm, acc):
        cid = lax.axis_index("c")
        sid = lax.axis_index("s")

        pltpu.sync_copy(cols_h.at[sid], cols_vm)
        pltpu.sync_copy(rows_h.at[sid], rows_vm)
        pltpu.sync_copy(vals_h.at[sid], vals_vm)

        def adjust_cols(off):
            def r_body(r, carry):
                for c8 in range(CHUNK // LANES):
                    sl = pl.ds(c8 * LANES, LANES)
                    cols_vm[r, sl] = cols_vm[r, sl] + off
                return carry
            lax.fori_loop(0, n_iters, r_body, 0)

        adjust_cols(cid * (2 * IN_F))

        for p in range(2):
            q = cid * 2 + p
            if p == 1:
                adjust_cols(jnp.int32(IN_F))

            sl_rows = pl.ds(sid * ROWS_PER_TILE, ROWS_PER_TILE)
            pltpu.sync_copy(bias_h.at[sl_rows], acc.at[sl_rows])

            plsc.subcore_barrier()

            def iter_body(i, carry):
                pltpu.sync_copy(x_h.at[cols_vm.at[i]], g_vm)

                def j16_body(j16, c2):
                    v16 = vals_vm[i, pl.ds(j16 * LANES, LANES)]
                    for l in range(LANES):
                        j = j16 * LANES + l
                        sv = _splat(v16, l)
                        for c4 in range(BCHUNK // LANES):
                            sl = pl.ds(c4 * LANES, LANES)
                            g_vm[j, sl] = g_vm[j, sl] * sv
                    return c2
                lax.fori_loop(0, CHUNK // LANES, j16_body, 0)

                pltpu.sync_copy(g_vm, acc.at[rows_vm.at[i]], add=True)
                return carry
            lax.fori_loop(0, n_iters, iter_body, 0)

            plsc.subcore_barrier()

            pltpu.sync_copy(
                acc.at[pl.ds(sid * ROWS_PER_TILE, ROWS_PER_TILE)],
                out_h.at[q, pl.ds(sid * ROWS_PER_TILE, ROWS_PER_TILE)])

    return sc_call


def kernel(inp, indices, values, bias):
    nnz = values.shape[0]
    per_tile = -(-nnz // (NS * CHUNK)) * CHUNK
    n_iters = per_tile // CHUNK
    total = per_tile * NS
    pad = total - nnz

    rows = indices[0].astype(jnp.int32)
    cols = indices[1].astype(jnp.int32)
    vals = values.astype(jnp.float32)
    zpad_i = jnp.zeros((pad,), jnp.int32)
    rows_p = jnp.concatenate([rows, zpad_i]).reshape(NS, n_iters, CHUNK)
    cols_p = jnp.concatenate([cols, zpad_i]).reshape(NS, n_iters, CHUNK)
    vals_p = jnp.concatenate([vals, jnp.zeros((pad,), jnp.float32)]
                             ).reshape(NS, n_iters, CHUNK)

    x = inp.reshape(4, BCHUNK, IN_F).transpose(0, 2, 1).reshape(4 * IN_F,
                                                                BCHUNK)
    bias64 = jnp.broadcast_to(bias.reshape(OUT_F, 1), (OUT_F, BCHUNK))
    out4 = _make_sc_call(n_iters)(x, cols_p, rows_p, vals_p, bias64)
    return out4.transpose(0, 2, 1).reshape(BATCH, OUT_F)


# speedup vs baseline: 5.4136x; 1.8977x over previous
import functools

import jax
import jax.numpy as jnp
from jax import lax
from jax.experimental import pallas as pl
from jax.experimental.pallas import tpu as pltpu
from jax.experimental.pallas import tpu_sc as plsc

OUT_F = 16384
IN_F = 16384
BATCH = 256

NC = 2
NS = 16
LANES = 16
CHUNK = 128
BCHUNK = 64
BS = 6
ROWS_PER_TILE = OUT_F // NS


def _splat(v16, l):
    idx = jnp.full((LANES, 1), l, jnp.int32)
    return lax.gather(
        v16, idx,
        dimension_numbers=lax.GatherDimensionNumbers(
            offset_dims=(), collapsed_slice_dims=(0,), start_index_map=(0,)),
        slice_sizes=(1,),
        mode=lax.GatherScatterMode.PROMISE_IN_BOUNDS)


def _make_sc_call(nb: int):
    mesh = plsc.VectorSubcoreMesh(core_axis_name="c", subcore_axis_name="s")

    @functools.partial(
        pl.kernel,
        out_type=jax.ShapeDtypeStruct((4, OUT_F, BCHUNK), jnp.float32),
        mesh=mesh,
        compiler_params=pltpu.CompilerParams(use_tc_tiling_on_sc=False),
        scratch_types=[
            pltpu.VMEM((2, CHUNK), jnp.int32),
            pltpu.VMEM((2, BS, CHUNK), jnp.int32),
            pltpu.VMEM((2, BS, CHUNK), jnp.int32),
            pltpu.VMEM((2, BS, CHUNK), jnp.float32),
            pltpu.VMEM((CHUNK, BCHUNK), jnp.float32),
            pltpu.VMEM((CHUNK, BCHUNK), jnp.float32),
            pltpu.VMEM((CHUNK, BCHUNK), jnp.float32),
            pltpu.VMEM((CHUNK, BCHUNK), jnp.float32),
            pltpu.VMEM_SHARED((CHUNK, BCHUNK), jnp.float32),
            pltpu.VMEM_SHARED((OUT_F, BCHUNK), jnp.float32),
            pltpu.SemaphoreType.DMA,
            pltpu.SemaphoreType.DMA,
            pltpu.SemaphoreType.DMA,
            pltpu.SemaphoreType.DMA,
            pltpu.SemaphoreType.DMA,
            pltpu.SemaphoreType.DMA,
        ],
    )
    def sc_call(x_h, colsS_h, head_h, rows_h, vals_h, bias_h, out_h,
                chead, cbuf, rbuf, vbuf, g0, g1, s0, s1, dummy, acc,
                sem_g0, sem_g1, sem_s0, sem_s1, sem_i0, sem_i1):
        cid = lax.axis_index("c")
        sid = lax.axis_index("s")
        g_bufs = (g0, g1)
        s_bufs = (s0, s1)
        sem_g = (sem_g0, sem_g1)
        sem_s = (sem_s0, sem_s1)
        sem_i = (sem_i0, sem_i1)
        sl_rows = pl.ds(sid * ROWS_PER_TILE, ROWS_PER_TILE)

        def scale(hb, j, b):
            gb, sb = g_bufs[b], s_bufs[b]

            def j16_body(j16, c2):
                v16 = vbuf[hb, j, pl.ds(j16 * LANES, LANES)]
                for l in range(LANES):
                    jj = j16 * LANES + l
                    sv = _splat(v16, l)
                    for c4 in range(BCHUNK // LANES):
                        sl = pl.ds(c4 * LANES, LANES)
                        sb[jj, sl] = gb[jj, sl] * sv
                return c2
            lax.fori_loop(0, CHUNK // LANES, j16_body, 0)

        def pass_body(p, carry):
            q = cid * 2 + p

            pltpu.sync_copy(bias_h.at[sl_rows], acc.at[sl_rows])
            plsc.subcore_barrier()

            pltpu.sync_copy(head_h.at[q, sid], chead)
            pltpu.async_copy(colsS_h.at[q, sid, 0], cbuf.at[0], sem_i[0])
            pltpu.async_copy(rows_h.at[sid, 0], rbuf.at[0], sem_i[0])
            pltpu.async_copy(vals_h.at[sid, 0], vbuf.at[0], sem_i[0])
            pltpu.async_copy(s_bufs[0], dummy, sem_s[0])
            pltpu.async_copy(s_bufs[1], dummy, sem_s[1])
            pltpu.async_copy(x_h.at[chead.at[0]], g_bufs[0], sem_g[0])
            pltpu.async_copy(x_h.at[chead.at[1]], g_bufs[1], sem_g[1])

            def superblock(ks, c):
                for hb in range(2):
                    kb = ks * 2 + hb
                    pltpu.make_async_copy(colsS_h.at[q, sid, 0],
                                          cbuf.at[hb], sem_i[hb]).wait()
                    pltpu.make_async_copy(rows_h.at[sid, 0],
                                          rbuf.at[hb], sem_i[hb]).wait()
                    pltpu.make_async_copy(vals_h.at[sid, 0],
                                          vbuf.at[hb], sem_i[hb]).wait()
                    for j in range(BS):
                        b = j % 2
                        pltpu.make_async_copy(x_h.at[cbuf.at[hb, j]],
                                              g_bufs[b], sem_g[b]).wait()
                        pltpu.make_async_copy(s_bufs[b],
                                              acc.at[rbuf.at[hb, j]],
                                              sem_s[b]).wait()
                        scale(hb, j, b)
                        pltpu.async_copy(x_h.at[cbuf.at[hb, j]],
                                         g_bufs[b], sem_g[b])
                        pltpu.async_copy(s_bufs[b], acc.at[rbuf.at[hb, j]],
                                         sem_s[b], add=True)
                        if j == 1:
                            kk = kb + 1
                            pltpu.async_copy(colsS_h.at[q, sid, kk],
                                             cbuf.at[1 - hb], sem_i[1 - hb])
                            pltpu.async_copy(rows_h.at[sid, kk],
                                             rbuf.at[1 - hb], sem_i[1 - hb])
                            pltpu.async_copy(vals_h.at[sid, kk],
                                             vbuf.at[1 - hb], sem_i[1 - hb])
                return c
            lax.fori_loop(0, nb // 2, superblock, 0)

            for b in range(2):
                pltpu.make_async_copy(s_bufs[b], acc.at[rbuf.at[0, 0]],
                                      sem_s[b]).wait()
                pltpu.make_async_copy(x_h.at[cbuf.at[0, 0]],
                                      g_bufs[b], sem_g[b]).wait()
            pltpu.make_async_copy(colsS_h.at[q, sid, 0], cbuf.at[0],
                                  sem_i[0]).wait()
            pltpu.make_async_copy(rows_h.at[sid, 0], rbuf.at[0],
                                  sem_i[0]).wait()
            pltpu.make_async_copy(vals_h.at[sid, 0], vbuf.at[0],
                                  sem_i[0]).wait()

            plsc.subcore_barrier()
            pltpu.sync_copy(acc.at[sl_rows], out_h.at[q, sl_rows])
            return carry
        lax.fori_loop(0, 2, pass_body, 0)

    return sc_call


def kernel(inp, indices, values, bias):
    nnz = values.shape[0]
    iters_per_tile = -(-nnz // (NS * CHUNK * BS)) * BS
    nb = iters_per_tile // BS
    per_tile = iters_per_tile * CHUNK
    pad = per_tile * NS - nnz

    rows = indices[0].astype(jnp.int32)
    cols = indices[1].astype(jnp.int32)
    vals = values.astype(jnp.float32)
    zpad_i = jnp.zeros((pad,), jnp.int32)
    rows_p = jnp.concatenate([rows, zpad_i]).reshape(NS, per_tile)
    cols_p = jnp.concatenate([cols, zpad_i]).reshape(NS, per_tile)
    vals_p = jnp.concatenate([vals, jnp.zeros((pad,), jnp.float32)]
                             ).reshape(NS, per_tile)

    blk_pad = (nb + 1) * BS * CHUNK - per_tile
    rows_hb = jnp.pad(rows_p, ((0, 0), (0, blk_pad))
                      ).reshape(NS, nb + 1, BS, CHUNK)
    vals_hb = jnp.pad(vals_p, ((0, 0), (0, blk_pad))
                      ).reshape(NS, nb + 1, BS, CHUNK)
    cols_shift = jnp.pad(cols_p[:, 2 * CHUNK:],
                         ((0, 0), (0, 2 * CHUNK + blk_pad)))
    offs = (jnp.arange(4, dtype=jnp.int32) * IN_F)[:, None, None]
    colsS = (cols_shift[None] + offs).reshape(4, NS, nb + 1, BS, CHUNK)
    head = (cols_p[None, :, :2 * CHUNK] + offs).reshape(4, NS, 2, CHUNK)

    x = inp.reshape(4, BCHUNK, IN_F).transpose(0, 2, 1).reshape(4 * IN_F,
                                                                BCHUNK)
    bias64 = jnp.broadcast_to(bias.reshape(OUT_F, 1), (OUT_F, BCHUNK))
    out4 = _make_sc_call(nb)(x, colsS, head, rows_hb, vals_hb, bias64)
    return out4.transpose(0, 2, 1).reshape(BATCH, OUT_F)
